# baseline (speedup 1.0000x reference)
import jax
import jax.numpy as jnp
from jax.experimental import pallas as pl
from jax.experimental.pallas import tpu as pltpu


def _slab_add_kernel(s_ref, cache_ref, contrib_ref, out_ref):
    del s_ref
    out_ref[...] = cache_ref[...] + contrib_ref[...]


def kernel(k_cache, k, step):
    B, H, S, D = k_cache.shape
    Q = k.shape[-2]
    start = jnp.clip(jnp.asarray(step, jnp.int32) - Q, 0, S - Q)
    s0 = start // Q
    off = start - s0 * Q

    contrib = jax.lax.dynamic_update_slice_in_dim(
        jnp.zeros((B, H, 2 * Q, D), k.dtype), k, off, axis=2)

    n_blocks = S // Q

    def cache_index(p, s_ref):
        idx = s_ref[0] // Q + p
        return (0, 0, jnp.where(idx >= n_blocks, 0, idx), 0)

    def contrib_index(p, s_ref):
        del s_ref
        return (0, 0, p, 0)

    cache_spec = pl.BlockSpec((B, H, Q, D), cache_index)
    contrib_spec = pl.BlockSpec((B, H, Q, D), contrib_index)

    grid_spec = pltpu.PrefetchScalarGridSpec(
        num_scalar_prefetch=1,
        grid=(2,),
        in_specs=[cache_spec, contrib_spec],
        out_specs=cache_spec,
    )
    return pl.pallas_call(
        _slab_add_kernel,
        grid_spec=grid_spec,
        out_shape=jax.ShapeDtypeStruct(k_cache.shape, k_cache.dtype),
        input_output_aliases={1: 0},
    )(start.reshape(1), k_cache, contrib)


# speedup vs baseline: 1.0958x; 1.0958x over previous
import jax
import jax.numpy as jnp
from jax.experimental import pallas as pl
from jax.experimental.pallas import tpu as pltpu


def _make_slab_kernel(Q):
    def _slab_add_kernel(s_ref, cache_ref, k_ref, out_ref):
        p = pl.program_id(0)
        start = s_ref[0]
        off = start - (start // Q) * Q
        kb = k_ref[...]
        rolled = pltpu.roll(kb, off, axis=1)
        r = jax.lax.broadcasted_iota(jnp.int32, kb.shape, 1)
        mask = jnp.logical_xor(r >= off, p != 0)
        out_ref[...] = cache_ref[...] + jnp.where(mask, rolled, 0.0)
    return _slab_add_kernel


def kernel(k_cache, k, step):
    B, H, S, D = k_cache.shape
    Q = k.shape[-2]
    BH = B * H
    start = jnp.clip(jnp.asarray(step, jnp.int32) - Q, 0, S - Q)

    kc = k_cache.reshape(BH, S, D)
    kk = k.reshape(BH, Q, D)
    n_blocks = S // Q

    def cache_index(p, s_ref):
        idx = s_ref[0] // Q + p
        return (0, jnp.where(idx >= n_blocks, 0, idx), 0)

    def k_index(p, s_ref):
        del p, s_ref
        return (0, 0, 0)

    cache_spec = pl.BlockSpec((BH, Q, D), cache_index)
    k_spec = pl.BlockSpec((BH, Q, D), k_index)

    grid_spec = pltpu.PrefetchScalarGridSpec(
        num_scalar_prefetch=1,
        grid=(2,),
        in_specs=[cache_spec, k_spec],
        out_specs=cache_spec,
    )
    out = pl.pallas_call(
        _make_slab_kernel(Q),
        grid_spec=grid_spec,
        out_shape=jax.ShapeDtypeStruct(kc.shape, kc.dtype),
        input_output_aliases={1: 0},
    )(start.reshape(1), kc, kk)
    return out.reshape(B, H, S, D)
